# baseline (speedup 1.0000x reference)
import functools

import jax
import jax.numpy as jnp
from jax import lax
from jax.experimental import pallas as pl
from jax.experimental.pallas import tpu as pltpu
from jax.experimental.pallas import tpu_sc as plsc

B, NS, NT, HID = 16, 4, 512, 128
HB = B // 2
HROWS = HB * NS * NT
NC, NSUB = 2, 16
CHUNK = 128
NCHUNK = NS
L = 16
EPS = 1e-12


def _sc_gather(ids_hbm, word_hbm, sent_hbm, out_hbm,
               idx_v, sidx_v, b0, b1, b2, b3, sbuf,
               g0, g1, g2, g3, w0, w1, w2, w3, ssem):
    bufs = [b0, b1, b2, b3]
    gsem = [g0, g1, g2, g3]
    wsem = [w0, w1, w2, w3]
    wid = lax.axis_index("s") * NC + lax.axis_index("c")
    c = lax.bitwise_and(wid, 3)
    g = lax.shift_right_logical(wid, 2)
    czero = c == 0
    lanes = lax.iota(jnp.int32, L)
    zer = jnp.zeros((L,), jnp.int32)

    pltpu.sync_copy(ids_hbm.at[g, :, pl.ds(c * CHUNK, CHUNK)], idx_v)
    sidx_v[...] = plsc.load_gather(
        idx_v, [lax.bitwise_and(lanes, 3), zer])
    scp = pltpu.async_copy(sent_hbm.at[sidx_v], sbuf, ssem)

    gh = [pltpu.async_copy(word_hbm.at[idx_v.at[k]], bufs[k], gsem[k])
          for k in range(NCHUNK)]
    scp.wait()

    wh = [None] * NCHUNK
    for k in range(NCHUNK):
        gh[k].wait()
        buf = bufs[k]
        kvec = zer + k
        for m in range(HID // L):
            cols = lanes + m * L
            wrow = plsc.load_gather(buf, [zer, cols])
            srow = plsc.load_gather(sbuf, [kvec, cols])
            plsc.store_scatter(buf, [zer, cols],
                               jnp.where(czero, srow, wrow))
        row_base = (NS * g + k) * NT + c * CHUNK
        wh[k] = pltpu.async_copy(buf, out_hbm.at[pl.ds(row_base, CHUNK)],
                                 wsem[k])
    for k in range(NCHUNK):
        wh[k].wait()


@functools.lru_cache(maxsize=None)
def _sc_gather_call():
    return pl.kernel(
        _sc_gather,
        out_type=jax.ShapeDtypeStruct((HROWS, HID), jnp.float32),
        mesh=plsc.VectorSubcoreMesh(
            core_axis_name="c", subcore_axis_name="s",
            num_cores=NC, num_subcores=NSUB),
        compiler_params=pltpu.CompilerParams(needs_layout_passes=False),
        scratch_types=(
            [pltpu.VMEM((NS, CHUNK), jnp.int32),
             pltpu.VMEM((L,), jnp.int32)]
            + [pltpu.VMEM((CHUNK, HID), jnp.float32)] * NCHUNK
            + [pltpu.VMEM((L, HID), jnp.float32)]
            + [pltpu.SemaphoreType.DMA] * (2 * NCHUNK + 1)
        ),
    )


def _tc_ln(x_ref, pos_ref, g_ref, b_ref, o_ref):
    x = x_ref[...]
    pos = pos_ref[...]
    t = lax.broadcasted_iota(jnp.int32, (NT, 1), 0)
    pos = jnp.where(t > 0, pos, 0.0)
    x = x + pos[None]
    u = jnp.mean(x, axis=-1, keepdims=True)
    d = x - u
    s = jnp.mean(d * d, axis=-1, keepdims=True)
    xn = d * lax.rsqrt(s + EPS)
    o_ref[...] = xn * g_ref[...] + b_ref[...]


def _tc_ln_call(gathered, pos_table, gamma2, beta2):
    nsent = HB * NS
    R = 8
    return pl.pallas_call(
        _tc_ln,
        grid=(nsent // R,),
        in_specs=[
            pl.BlockSpec((R, NT, HID), lambda i: (i, 0, 0)),
            pl.BlockSpec((NT, HID), lambda i: (0, 0)),
            pl.BlockSpec((1, HID), lambda i: (0, 0)),
            pl.BlockSpec((1, HID), lambda i: (0, 0)),
        ],
        out_specs=pl.BlockSpec((R, NT, HID), lambda i: (i, 0, 0)),
        out_shape=jax.ShapeDtypeStruct((nsent, NT, HID), jnp.float32),
    )(gathered.reshape(nsent, NT, HID), pos_table, gamma2, beta2)


def kernel(input_ids, word_table, pos_table, sent_table, gamma, beta):
    sc = _sc_gather_call()
    ga = sc(input_ids[:HB], word_table, sent_table)
    gb = sc(input_ids[HB:], word_table, sent_table)
    gamma2 = gamma.reshape(1, HID)
    beta2 = beta.reshape(1, HID)
    oa = _tc_ln_call(ga, pos_table, gamma2, beta2)
    ob = _tc_ln_call(gb, pos_table, gamma2, beta2)
    out = jnp.concatenate([oa, ob], axis=0)
    return out.reshape(B, NS, NT, HID)


# speedup vs baseline: 1.2884x; 1.2884x over previous
import functools

import jax
import jax.numpy as jnp
from jax import lax
from jax.experimental import pallas as pl
from jax.experimental.pallas import tpu as pltpu
from jax.experimental.pallas import tpu_sc as plsc

B, NS, NT, HID = 16, 4, 512, 128
ROWS = B * NS * NT
NC, NSUB = 2, 16
NW = NC * NSUB
CHUNK = 128
NCHUNK = 8
NBUF = 7
L = 16
EPS = 1e-12


def _sc_gather(ids_hbm, word_hbm, sent_hbm, out_hbm,
               idx_v, sidx_v, b0, b1, b2, b3, b4, b5, b6, sbuf,
               g0, g1, g2, g3, g4, g5, g6,
               w0, w1, w2, w3, w4, w5, w6, ssem):
    bufs = [b0, b1, b2, b3, b4, b5, b6]
    gsem = [g0, g1, g2, g3, g4, g5, g6]
    wsem = [w0, w1, w2, w3, w4, w5, w6]
    wid = lax.axis_index("s") * NC + lax.axis_index("c")
    c = lax.bitwise_and(wid, 3)
    g = lax.shift_right_logical(wid, 2)
    czero = c == 0
    lanes = lax.iota(jnp.int32, L)
    zer = jnp.zeros((L,), jnp.int32)

    pltpu.sync_copy(ids_hbm.at[pl.ds(2 * g, 2), :, pl.ds(c * CHUNK, CHUNK)],
                    idx_v)
    k_lane = lax.bitwise_and(lanes, 7)
    sidx_v[...] = plsc.load_gather(
        idx_v, [lax.shift_right_logical(k_lane, 2),
                lax.bitwise_and(k_lane, 3), zer])
    scp = pltpu.async_copy(sent_hbm.at[sidx_v], sbuf, ssem)

    gh = [pltpu.async_copy(word_hbm.at[idx_v.at[k // 4, k % 4]],
                           bufs[k], gsem[k]) for k in range(NBUF)]
    scp.wait()

    wh = [None] * NBUF
    for k in range(NCHUNK):
        b = k % NBUF
        buf = bufs[b]
        gh[b].wait()
        kvec = zer + k
        for m in range(HID // L):
            cols = lanes + m * L
            wrow = plsc.load_gather(buf, [zer, cols])
            srow = plsc.load_gather(sbuf, [kvec, cols])
            plsc.store_scatter(buf, [zer, cols],
                               jnp.where(czero, srow, wrow))
        row_base = (8 * g + k) * NT + c * CHUNK
        wh[b] = pltpu.async_copy(buf, out_hbm.at[pl.ds(row_base, CHUNK)],
                                 wsem[b])
        if k + NBUF < NCHUNK:
            wh[b].wait()
            kk = k + NBUF
            gh[b] = pltpu.async_copy(word_hbm.at[idx_v.at[kk // 4, kk % 4]],
                                     bufs[b], gsem[b])
    for b in range(NBUF):
        if wh[b] is not None:
            wh[b].wait()


@functools.lru_cache(maxsize=None)
def _sc_gather_call():
    return pl.kernel(
        _sc_gather,
        out_type=jax.ShapeDtypeStruct((ROWS, HID), jnp.float32),
        mesh=plsc.VectorSubcoreMesh(
            core_axis_name="c", subcore_axis_name="s",
            num_cores=NC, num_subcores=NSUB),
        compiler_params=pltpu.CompilerParams(needs_layout_passes=False),
        scratch_types=(
            [pltpu.VMEM((2, NS, CHUNK), jnp.int32),
             pltpu.VMEM((L,), jnp.int32)]
            + [pltpu.VMEM((CHUNK, HID), jnp.float32)] * NBUF
            + [pltpu.VMEM((L, HID), jnp.float32)]
            + [pltpu.SemaphoreType.DMA] * (2 * NBUF + 1)
        ),
    )


def _tc_ln(x_ref, pos_ref, g_ref, b_ref, o_ref):
    x = x_ref[...]
    pos = pos_ref[...]
    t = lax.broadcasted_iota(jnp.int32, (NT, 1), 0)
    pos = jnp.where(t > 0, pos, 0.0)
    x = x + pos[None]
    u = jnp.mean(x, axis=-1, keepdims=True)
    d = x - u
    s = jnp.mean(d * d, axis=-1, keepdims=True)
    xn = d * lax.rsqrt(s + EPS)
    o_ref[...] = xn * g_ref[...] + b_ref[...]


def kernel(input_ids, word_table, pos_table, sent_table, gamma, beta):
    gathered = _sc_gather_call()(input_ids, word_table, sent_table)

    nsent = B * NS
    R = 8
    out = pl.pallas_call(
        _tc_ln,
        grid=(nsent // R,),
        in_specs=[
            pl.BlockSpec((R, NT, HID), lambda i: (i, 0, 0)),
            pl.BlockSpec((NT, HID), lambda i: (0, 0)),
            pl.BlockSpec((1, HID), lambda i: (0, 0)),
            pl.BlockSpec((1, HID), lambda i: (0, 0)),
        ],
        out_specs=pl.BlockSpec((R, NT, HID), lambda i: (i, 0, 0)),
        out_shape=jax.ShapeDtypeStruct((nsent, NT, HID), jnp.float32),
    )(gathered.reshape(nsent, NT, HID), pos_table,
      gamma.reshape(1, HID), beta.reshape(1, HID))
    return out.reshape(B, NS, NT, HID)
